# baseline (speedup 1.0000x reference)
import jax
import jax.numpy as jnp
from jax.experimental import pallas as pl

_MEL = 80
_HID = 384
_K = 5
_STRIDE = 8
_NB = 6
_VQB = 1024
_VQD = 256
_B = 16
_T = 2048
_T2 = _T // _STRIDE

_TT = 512
_HALO = 16
_WIN = _TT + 2 * _HALO
_NT = _T // _TT

_PREC = jax.lax.Precision.HIGHEST


def _mm(a, b):
    return jax.lax.dot_general(a, b, (((1,), (0,)), ((), ())),
                               precision=_PREC,
                               preferred_element_type=jnp.float32)


def _conv(x, Wt, b):
    T = x.shape[0]
    xp = jnp.pad(x, ((_K // 2, _K // 2), (0, 0)))
    out = None
    for k in range(_K):
        part = _mm(xp[k:k + T], Wt[k])
        out = part if out is None else out + part
    return out + b


def _stage1_body(mel_ref, Win_ref, bin_ref, Wpre_ref, bpre_ref, hmid_ref):
    t = pl.program_id(1)
    x = mel_ref[0, pl.ds(t * _TT, _WIN), :]

    j = jax.lax.broadcasted_iota(jnp.int32, (_WIN, 1), 0)
    p = t * _TT - _HALO + j
    mask = jnp.logical_and(p >= 0, p < _T).astype(jnp.float32)

    h = mask * jnp.maximum(_conv(x, Win_ref[...], bin_ref[...]), 0.0)
    for i in range(_NB):
        h = mask * (h + jnp.maximum(_conv(h, Wpre_ref[i], bpre_ref[i:i + 1]),
                                    0.0))

    hc = h[_HALO:_HALO + _TT]
    hmid_ref[0] = jnp.max(hc.reshape(_TT // _STRIDE, _STRIDE, _HID), axis=1)


def _stage2_body(hmid_ref, Wpost_ref, bpost_ref, Wout_ref, bout_ref, cb_ref,
                 zq_ref, loss_ref):
    bidx = pl.program_id(0)
    h = hmid_ref[0]

    for i in range(_NB):
        h = h + jnp.maximum(_conv(h, Wpost_ref[i], bpost_ref[i:i + 1]), 0.0)

    ze = _conv(h, Wout_ref[...], bout_ref[...])

    cb = cb_ref[...]
    score = jax.lax.dot_general(
        ze, cb, (((1,), (1,)), ((), ())), precision=_PREC,
        preferred_element_type=jnp.float32)
    cbn = 0.5 * jnp.sum(cb * cb, axis=1)
    score = score - cbn[None, :]
    idx = jnp.argmax(score, axis=1)

    iota = jax.lax.broadcasted_iota(jnp.int32, (_T2, _VQB), 1)
    onehot = (iota == idx[:, None]).astype(jnp.float32)
    q = _mm(onehot, cb)

    zq_ref[0] = q
    err = ze - q
    s = jnp.sum(err * err)[None, None]
    loss_ref[...] = jnp.where(bidx == 0, s, loss_ref[...] + s)


def kernel(mel, W_in, b_in, W_pre, b_pre, W_post, b_post, W_out, b_out,
           codebook):
    Win_t = jnp.transpose(W_in, (2, 1, 0))
    Wpre_t = jnp.transpose(W_pre, (0, 3, 2, 1))
    Wpost_t = jnp.transpose(W_post, (0, 3, 2, 1))
    Wout_t = jnp.transpose(W_out, (2, 1, 0))
    bin2 = b_in[None, :]
    bout2 = b_out[None, :]
    mel_pad = jnp.pad(mel, ((0, 0), (_HALO, _HALO), (0, 0)))

    def full(a, ngrid):
        return pl.BlockSpec(a.shape, lambda *g: (0,) * a.ndim)

    hmid = pl.pallas_call(
        _stage1_body,
        grid=(_B, _NT),
        in_specs=[
            pl.BlockSpec((1, _T + 2 * _HALO, _MEL), lambda b, t: (b, 0, 0)),
            full(Win_t, 2), full(bin2, 2), full(Wpre_t, 2), full(b_pre, 2),
        ],
        out_specs=pl.BlockSpec((1, _TT // _STRIDE, _HID),
                               lambda b, t: (b, t, 0)),
        out_shape=jax.ShapeDtypeStruct((_B, _T2, _HID), jnp.float32),
    )(mel_pad, Win_t, bin2, Wpre_t, b_pre)

    zq, loss_sum = pl.pallas_call(
        _stage2_body,
        grid=(_B,),
        in_specs=[
            pl.BlockSpec((1, _T2, _HID), lambda b: (b, 0, 0)),
            full(Wpost_t, 1), full(b_post, 1), full(Wout_t, 1),
            full(bout2, 1), full(codebook, 1),
        ],
        out_specs=[
            pl.BlockSpec((1, _T2, _VQD), lambda b: (b, 0, 0)),
            pl.BlockSpec((1, 1), lambda b: (0, 0)),
        ],
        out_shape=[
            jax.ShapeDtypeStruct((_B, _T2, _VQD), jnp.float32),
            jax.ShapeDtypeStruct((1, 1), jnp.float32),
        ],
    )(hmid, Wpost_t, b_post, Wout_t, bout2, codebook)

    loss = loss_sum[0, 0] / jnp.float32(_B * _T2 * _VQD)
    return (zq, loss, loss)


# speedup vs baseline: 1.3870x; 1.3870x over previous
import jax
import jax.numpy as jnp
from jax.experimental import pallas as pl

_MEL = 80
_HID = 384
_K = 5
_STRIDE = 8
_NB = 6
_VQB = 1024
_VQD = 256
_B = 16
_T = 2048
_T2 = _T // _STRIDE

_TT = 512
_HALO = 16
_WIN = _TT + 2 * _HALO
_NT = _T // _TT

_PREC = jax.lax.Precision.HIGHEST


def _mm(a, b):
    return jax.lax.dot_general(a, b, (((1,), (0,)), ((), ())),
                               precision=_PREC,
                               preferred_element_type=jnp.float32)


def _conv(x, Wt, b):
    T = x.shape[0]
    xp = jnp.pad(x, ((_K // 2, _K // 2), (0, 0)))
    out = None
    for k in range(_K):
        part = _mm(xp[k:k + T], Wt[k])
        out = part if out is None else out + part
    return out + b


def _res_layer(W_ref, b_ref, i, h, mask):
    T = h.shape[0]
    xp = jnp.pad(h, ((_K // 2, _K // 2), (0, 0)))
    out = None
    for k in range(_K):
        part = _mm(xp[k:k + T], W_ref[i, k])
        out = part if out is None else out + part
    out = out + b_ref[pl.ds(i, 1)]
    out = h + jnp.maximum(out, 0.0)
    if mask is not None:
        out = mask * out
    return out


def _stage1_body(mel_ref, Win_ref, bin_ref, Wpre_ref, bpre_ref, hmid_ref):
    t = pl.program_id(1)
    x = mel_ref[0, pl.ds(t * _TT, _WIN), :]

    j = jax.lax.broadcasted_iota(jnp.int32, (_WIN, 1), 0)
    p = t * _TT - _HALO + j
    mask = jnp.logical_and(p >= 0, p < _T).astype(jnp.float32)

    h = mask * jnp.maximum(_conv(x, Win_ref[...], bin_ref[...]), 0.0)
    h = jax.lax.fori_loop(
        0, _NB, lambda i, hh: _res_layer(Wpre_ref, bpre_ref, i, hh, mask), h)

    hc = h[_HALO:_HALO + _TT]
    hmid_ref[0] = jnp.max(hc.reshape(_TT // _STRIDE, _STRIDE, _HID), axis=1)


def _stage2_body(hmid_ref, Wpost_ref, bpost_ref, Wout_ref, bout_ref, cb_ref,
                 zq_ref, loss_ref):
    bidx = pl.program_id(0)
    h = hmid_ref[0]

    h = jax.lax.fori_loop(
        0, _NB, lambda i, hh: _res_layer(Wpost_ref, bpost_ref, i, hh, None), h)

    ze = _conv(h, Wout_ref[...], bout_ref[...])

    cb = cb_ref[...]
    score = jax.lax.dot_general(
        ze, cb, (((1,), (1,)), ((), ())), precision=_PREC,
        preferred_element_type=jnp.float32)
    cbn = 0.5 * jnp.sum(cb * cb, axis=1)
    score = score - cbn[None, :]
    idx = jnp.argmax(score, axis=1)

    iota = jax.lax.broadcasted_iota(jnp.int32, (_T2, _VQB), 1)
    onehot = (iota == idx[:, None]).astype(jnp.float32)
    q = _mm(onehot, cb)

    zq_ref[0] = q
    err = ze - q
    s = jnp.sum(err * err)[None, None]
    loss_ref[...] = jnp.where(bidx == 0, s, loss_ref[...] + s)


def kernel(mel, W_in, b_in, W_pre, b_pre, W_post, b_post, W_out, b_out,
           codebook):
    Win_t = jnp.transpose(W_in, (2, 1, 0))
    Wpre_t = jnp.transpose(W_pre, (0, 3, 2, 1))
    Wpost_t = jnp.transpose(W_post, (0, 3, 2, 1))
    Wout_t = jnp.transpose(W_out, (2, 1, 0))
    bin2 = b_in[None, :]
    bout2 = b_out[None, :]
    mel_pad = jnp.pad(mel, ((0, 0), (_HALO, _HALO), (0, 0)))

    def full(a, ngrid):
        return pl.BlockSpec(a.shape, lambda *g: (0,) * a.ndim)

    hmid = pl.pallas_call(
        _stage1_body,
        grid=(_B, _NT),
        in_specs=[
            pl.BlockSpec((1, _T + 2 * _HALO, _MEL), lambda b, t: (b, 0, 0)),
            full(Win_t, 2), full(bin2, 2), full(Wpre_t, 2), full(b_pre, 2),
        ],
        out_specs=pl.BlockSpec((1, _TT // _STRIDE, _HID),
                               lambda b, t: (b, t, 0)),
        out_shape=jax.ShapeDtypeStruct((_B, _T2, _HID), jnp.float32),
    )(mel_pad, Win_t, bin2, Wpre_t, b_pre)

    zq, loss_sum = pl.pallas_call(
        _stage2_body,
        grid=(_B,),
        in_specs=[
            pl.BlockSpec((1, _T2, _HID), lambda b: (b, 0, 0)),
            full(Wpost_t, 1), full(b_post, 1), full(Wout_t, 1),
            full(bout2, 1), full(codebook, 1),
        ],
        out_specs=[
            pl.BlockSpec((1, _T2, _VQD), lambda b: (b, 0, 0)),
            pl.BlockSpec((1, 1), lambda b: (0, 0)),
        ],
        out_shape=[
            jax.ShapeDtypeStruct((_B, _T2, _VQD), jnp.float32),
            jax.ShapeDtypeStruct((1, 1), jnp.float32),
        ],
    )(hmid, Wpost_t, b_post, Wout_t, bout2, codebook)

    loss = loss_sum[0, 0] / jnp.float32(_B * _T2 * _VQD)
    return (zq, loss, loss)
